# baseline (speedup 1.0000x reference)
import jax
import jax.numpy as jnp
from jax.experimental import pallas as pl

_N = 10000
_D = 128
_BM = 200
_NB = _N // _BM


def _hop1_body(a_ref, x_ref, h1_ref):
    h1_ref[...] = jnp.dot(a_ref[...], x_ref[...],
                          preferred_element_type=jnp.float32)


def _hop2_body(a_ref, h1_ref, w1_ref, b1_ref, w2_ref, b2_ref, out_ref):
    h2 = jnp.dot(a_ref[...], h1_ref[...], preferred_element_type=jnp.float32)
    hid = jnp.maximum(
        jnp.dot(h2, w1_ref[...].T, preferred_element_type=jnp.float32)
        + b1_ref[...], 0.0)
    row = jnp.sum(hid * w2_ref[...], axis=1) + b2_ref[0, 0]
    out_ref[...] = row.reshape(1, 1, _BM)


def kernel(x, adj_gcn, W1, b1, W2, b2):
    h1 = pl.pallas_call(
        _hop1_body,
        grid=(_NB,),
        in_specs=[
            pl.BlockSpec((_BM, _N), lambda i: (i, 0)),
            pl.BlockSpec((_N, _D), lambda i: (0, 0)),
        ],
        out_specs=pl.BlockSpec((_BM, _D), lambda i: (i, 0)),
        out_shape=jax.ShapeDtypeStruct((_N, _D), jnp.float32),
    )(adj_gcn, x)

    out3 = pl.pallas_call(
        _hop2_body,
        grid=(_NB,),
        in_specs=[
            pl.BlockSpec((_BM, _N), lambda i: (i, 0)),
            pl.BlockSpec((_N, _D), lambda i: (0, 0)),
            pl.BlockSpec((_D, _D), lambda i: (0, 0)),
            pl.BlockSpec((1, _D), lambda i: (0, 0)),
            pl.BlockSpec((1, _D), lambda i: (0, 0)),
            pl.BlockSpec((1, 1), lambda i: (0, 0)),
        ],
        out_specs=pl.BlockSpec((1, 1, _BM), lambda i: (i, 0, 0)),
        out_shape=jax.ShapeDtypeStruct((_NB, 1, _BM), jnp.float32),
    )(adj_gcn, h1, W1, b1.reshape(1, _D), W2.reshape(1, _D),
      jnp.asarray(b2).reshape(1, 1))

    return out3.reshape(_N)
